# baseline (speedup 1.0000x reference)
import functools

import jax
import jax.numpy as jnp
from jax import lax
from jax.experimental import pallas as pl
from jax.experimental.pallas import tpu as pltpu
from jax.experimental.pallas import tpu_sc as plsc

_N = 50000
_E = 800000
_HID = 64
_HALF = 32
_OUT = 128
_L = 4
_CH = 128
_SUP = 2
_NSUB = 16
_NCHUNK = 392
_NSUP = _NCHUNK // _SUP
_EPT = _NCHUNK * _CH
_EP = _NSUB * _EPT
_NROWS = _EP // _CH
_N2 = 50048
_RPT = _N2 // _NSUB
_WPT = 3128
_WLAST = _N - 15 * _WPT
_BN = 2000
_BE = 2048

_MESH = plsc.VectorSubcoreMesh(core_axis_name="c", subcore_axis_name="s")
_SC_PARAMS = pltpu.CompilerParams(use_tc_tiling_on_sc=False)


def _in_body(xb, wb, bb, ob):
    ob[...] = jnp.dot(xb[...], wb[...], preferred_element_type=jnp.float32) + bb[...]


def _h_in(xp, W, b):
    return pl.pallas_call(
        _in_body,
        grid=(_N // _BN,),
        in_specs=[
            pl.BlockSpec((_BN, 8), lambda i: (i, 0)),
            pl.BlockSpec((8, _HID), lambda i: (0, 0)),
            pl.BlockSpec((1, _HID), lambda i: (0, 0)),
        ],
        out_specs=pl.BlockSpec((_BN, _HID), lambda i: (i, 0)),
        out_shape=jax.ShapeDtypeStruct((_N, _HID), jnp.float32),
    )(xp, W, b)


_E16 = _E // 16
_EP16 = _EP // 16
_BE16 = 1024


def _c_body(eb, wb, bb, ob):
    i = pl.program_id(0)
    rows = i * _BE16 + lax.broadcasted_iota(jnp.int32, (_BE16, 1), 0)
    m = jnp.dot(eb[...], wb[0, 0], preferred_element_type=jnp.float32) + bb[0, 0]
    ob[...] = jnp.where(rows < _E16, m, -1e30).reshape(_BE16 * 512)


def _c_all(ea16, W16, b16):
    nblk = _EP16 // _BE16
    return pl.pallas_call(
        _c_body,
        grid=(nblk, _L, 2),
        in_specs=[
            pl.BlockSpec((_BE16, 64), lambda e, l, c: (jnp.minimum(e, _E16 // _BE16), 0)),
            pl.BlockSpec((1, 1, 64, 512), lambda e, l, c: (l, c, 0, 0)),
            pl.BlockSpec((1, 1, 1, 512), lambda e, l, c: (l, c, 0, 0)),
        ],
        out_specs=pl.BlockSpec((_BE16 * 512,), lambda e, l, c: ((l * 2 + c) * nblk + e,)),
        out_shape=jax.ShapeDtypeStruct((_L * 2 * _EP * _HALF,), jnp.float32),
    )(ea16, W16, b16)


def _ab_first_body(hb, wab, wbb, ao, bo):
    h = hb[...]
    ao[...] = jnp.dot(h, wab[0], preferred_element_type=jnp.float32)
    bo[...] = jnp.dot(h, wbb[0], preferred_element_type=jnp.float32)


def _ab_first(h, Wa, Wb):
    nb = _N // _BN
    return pl.pallas_call(
        _ab_first_body,
        grid=(2, nb),
        in_specs=[
            pl.BlockSpec((_BN, _HID), lambda c, i: (i, 0)),
            pl.BlockSpec((1, _HID, _HALF), lambda c, i: (c, 0, 0)),
            pl.BlockSpec((1, _HID, _HALF), lambda c, i: (c, 0, 0)),
        ],
        out_specs=[
            pl.BlockSpec((_BN, _HALF), lambda c, i: (c * nb + i, 0)),
            pl.BlockSpec((_BN, _HALF), lambda c, i: (c * nb + i, 0)),
        ],
        out_shape=[
            jax.ShapeDtypeStruct((2 * _N, _HALF), jnp.float32),
            jax.ShapeDtypeStruct((2 * _N, _HALF), jnp.float32),
        ],
    )(h, Wa, Wb)


def _ab_body(hb, gb, cb, wab, wbb, ho, ao, bo):
    inv = 1.0 / jnp.maximum(cb[...][:, :1], 1.0)
    hn = hb[...] + jnp.concatenate([gb[0], gb[1]], axis=1) * inv
    ho[...] = hn
    ao[...] = jnp.dot(hn, wab[0], preferred_element_type=jnp.float32)
    bo[...] = jnp.dot(hn, wbb[0], preferred_element_type=jnp.float32)


def _ab_next(h, agg, cnt, Wa, Wb):
    nb = _N // _BN
    return pl.pallas_call(
        _ab_body,
        grid=(2, nb),
        in_specs=[
            pl.BlockSpec((_BN, _HID), lambda c, i: (i, 0)),
            pl.BlockSpec((2, _BN, _HALF), lambda c, i: (0, i, 0)),
            pl.BlockSpec((_BN, 16), lambda c, i: (i, 0)),
            pl.BlockSpec((1, _HID, _HALF), lambda c, i: (c, 0, 0)),
            pl.BlockSpec((1, _HID, _HALF), lambda c, i: (c, 0, 0)),
        ],
        out_specs=[
            pl.BlockSpec((_BN, _HID), lambda c, i: (i, 0)),
            pl.BlockSpec((_BN, _HALF), lambda c, i: (c * nb + i, 0)),
            pl.BlockSpec((_BN, _HALF), lambda c, i: (c * nb + i, 0)),
        ],
        out_shape=[
            jax.ShapeDtypeStruct((_N, _HID), jnp.float32),
            jax.ShapeDtypeStruct((2 * _N, _HALF), jnp.float32),
            jax.ShapeDtypeStruct((2 * _N, _HALF), jnp.float32),
        ],
    )(h, agg, cnt, Wa, Wb)


def _out_body(hb, gb, cb, wb, bb, ob):
    inv = 1.0 / jnp.maximum(cb[...][:, :1], 1.0)
    hn = hb[...] + jnp.concatenate([gb[0], gb[1]], axis=1) * inv
    ob[...] = jnp.dot(hn, wb[...], preferred_element_type=jnp.float32) + bb[...]


def _out_proj(h, agg, cnt, W, b):
    return pl.pallas_call(
        _out_body,
        grid=(_N // _BN,),
        in_specs=[
            pl.BlockSpec((_BN, _HID), lambda i: (i, 0)),
            pl.BlockSpec((2, _BN, _HALF), lambda i: (0, i, 0)),
            pl.BlockSpec((_BN, 16), lambda i: (i, 0)),
            pl.BlockSpec((_HID, _OUT), lambda i: (0, 0)),
            pl.BlockSpec((1, _OUT), lambda i: (0, 0)),
        ],
        out_specs=pl.BlockSpec((_BN, _OUT), lambda i: (i, 0)),
        out_shape=jax.ShapeDtypeStruct((_N, _OUT), jnp.float32),
    )(h, agg, cnt, W, b)


def _hist_kernel():
    @functools.partial(
        pl.kernel,
        out_type=jax.ShapeDtypeStruct((_N, 16), jnp.float32),
        mesh=_MESH,
        compiler_params=_SC_PARAMS,
        scratch_types=[
            pltpu.VMEM((_SUP, _CH), jnp.int32),
            pltpu.VMEM((_CH, 16), jnp.float32),
            pltpu.VMEM_SHARED((_N2, 16), jnp.float32),
            pltpu.SemaphoreType.DMA,
        ],
    )
    def hist(dsth, Z16, ones, out, di, ones_v, accc, ss):
        cid = lax.axis_index("c")
        sid = lax.axis_index("s")

        @pl.when(cid == 0)
        def _():
            pltpu.sync_copy(
                Z16.at[pl.ds(sid * _RPT, _RPT)],
                accc.at[pl.ds(sid * _RPT, _RPT)],
            )
            pltpu.sync_copy(ones, ones_v)
            plsc.subcore_barrier()

            def step(u, carry):
                r0 = sid * _NCHUNK + u * _SUP
                pltpu.sync_copy(dsth.at[pl.ds(r0, _SUP)], di)
                cps = [
                    pltpu.async_copy(ones_v, accc.at[di.at[j]], ss, add=True)
                    for j in range(_SUP)
                ]
                for g in cps:
                    g.wait()
                return carry

            lax.fori_loop(0, _NSUP, step, 0)
            plsc.subcore_barrier()

            @pl.when(sid < _NSUB - 1)
            def _():
                pltpu.sync_copy(
                    accc.at[pl.ds(sid * _WPT, _WPT)],
                    out.at[pl.ds(sid * _WPT, _WPT)],
                )

            @pl.when(sid == _NSUB - 1)
            def _():
                pltpu.sync_copy(
                    accc.at[pl.ds(15 * _WPT, _WLAST)],
                    out.at[pl.ds(15 * _WPT, _WLAST)],
                )

    return hist


def _edge_kernel(l):
    @functools.partial(
        pl.kernel,
        out_type=jax.ShapeDtypeStruct((2, _N, _HALF), jnp.float32),
        mesh=_MESH,
        compiler_params=_SC_PARAMS,
        scratch_types=[
            pltpu.VMEM((_SUP, _CH), jnp.int32),
            pltpu.VMEM((_SUP, _CH), jnp.int32),
            pltpu.VMEM((_SUP, _CH), jnp.int32),
            pltpu.VMEM((_SUP, _CH), jnp.int32),
            pltpu.VMEM((_SUP, _CH), jnp.int32),
            pltpu.VMEM((_SUP, _CH), jnp.int32),
            pltpu.VMEM((_SUP * _CH, _HALF), jnp.float32),
            pltpu.VMEM((_SUP * _CH, _HALF), jnp.float32),
            pltpu.VMEM_SHARED((_N2, _HALF), jnp.float32),
            pltpu.SemaphoreType.DMA,
            pltpu.SemaphoreType.DMA,
            pltpu.SemaphoreType.DMA,
            pltpu.SemaphoreType.DMA,
            pltpu.SemaphoreType.DMA,
            pltpu.SemaphoreType.DMA,
            pltpu.SemaphoreType.DMA,
            pltpu.SemaphoreType.DMA,
        ],
    )
    def edge(srcp, dstp, A, B, C, Z, out,
             si0, si1, di0, di1, bi0, bi1, c0, c1, acc,
             mi0, mi1, mc0, mc1, mg0, mg1, ms0, ms1):
        cid = lax.axis_index("c")
        sid = lax.axis_index("s")
        SI, DI, BI = (si0, si1), (di0, di1), (bi0, bi1)
        CS = (c0, c1)
        MI, MC, MG, MS = (mi0, mi1), (mc0, mc1), (mg0, mg1), (ms0, ms1)
        off = jnp.full((16,), cid * _N, jnp.int32)

        pltpu.sync_copy(Z.at[pl.ds(sid * _RPT, _RPT)], acc.at[pl.ds(sid * _RPT, _RPT)])
        plsc.subcore_barrier()

        def issue_loads(i, b):
            r0 = sid * _NCHUNK + i * _SUP
            pltpu.async_copy(srcp.at[pl.ds(r0, _SUP)], SI[b], MI[b])
            pltpu.async_copy(dstp.at[pl.ds(r0, _SUP)], DI[b], MI[b])
            pltpu.async_copy(C.at[l, cid, pl.ds(r0 * _CH, _SUP * _CH)], CS[b], MC[b])

        def wait_loads(i, b):
            r0 = sid * _NCHUNK + i * _SUP
            pltpu.make_async_copy(srcp.at[pl.ds(r0, _SUP)], SI[b], MI[b]).wait()
            pltpu.make_async_copy(dstp.at[pl.ds(r0, _SUP)], DI[b], MI[b]).wait()
            pltpu.make_async_copy(C.at[l, cid, pl.ds(r0 * _CH, _SUP * _CH)], CS[b], MC[b]).wait()

        def offs(b):
            for k in range(_SUP):
                for g in range(_CH // 16):
                    s = pl.ds(g * 16, 16)
                    BI[b][k, s] = DI[b][k, s] + off
                    SI[b][k, s] = SI[b][k, s] + off

        def issue_gathers(b):
            for j in range(_SUP):
                pltpu.async_copy(A.at[SI[b].at[j]], CS[b].at[pl.ds(j * _CH, _CH)], MG[b], add=True)
                pltpu.async_copy(B.at[BI[b].at[j]], CS[b].at[pl.ds(j * _CH, _CH)], MG[b], add=True)

        def wait_gathers(b):
            for j in range(_SUP):
                pltpu.make_async_copy(A.at[SI[b].at[j]], CS[b].at[pl.ds(j * _CH, _CH)], MG[b]).wait()
                pltpu.make_async_copy(B.at[BI[b].at[j]], CS[b].at[pl.ds(j * _CH, _CH)], MG[b]).wait()

        def relu(b):
            def rows(rb, c2):
                for t in range(4):
                    r = rb * 4 + t
                    for f in (0, 16):
                        s = pl.ds(f, 16)
                        CS[b][r, s] = jnp.maximum(CS[b][r, s], 0.0)
                return c2

            lax.fori_loop(0, (_SUP * _CH) // 4, rows, 0)

        def issue_scatter(b):
            for j in range(_SUP):
                pltpu.async_copy(CS[b].at[pl.ds(j * _CH, _CH)], acc.at[DI[b].at[j]], MS[b], add=True)

        def wait_scatter(b):
            for j in range(_SUP):
                pltpu.make_async_copy(CS[b].at[pl.ds(j * _CH, _CH)], acc.at[DI[b].at[j]], MS[b]).wait()

        issue_loads(0, 0)
        issue_loads(1, 1)
        wait_loads(0, 0)
        offs(0)
        issue_gathers(0)

        def body(i2, carry):
            i = i2 * 2
            wait_loads(i + 1, 1)
            offs(1)
            issue_gathers(1)

            wait_gathers(0)
            relu(0)
            issue_scatter(0)
            wait_scatter(0)

            @pl.when(i + 2 < _NSUP)
            def _():
                issue_loads(i + 2, 0)

            wait_gathers(1)
            relu(1)
            issue_scatter(1)
            wait_scatter(1)

            @pl.when(i + 3 < _NSUP)
            def _():
                issue_loads(i + 3, 1)

            @pl.when(i + 2 < _NSUP)
            def _():
                wait_loads(i + 2, 0)
                offs(0)
                issue_gathers(0)

            return carry

        lax.fori_loop(0, _NSUP // 2, body, 0)
        plsc.subcore_barrier()

        @pl.when(sid < _NSUB - 1)
        def _():
            pltpu.sync_copy(
                acc.at[pl.ds(sid * _WPT, _WPT)],
                out.at[cid, pl.ds(sid * _WPT, _WPT)],
            )

        @pl.when(sid == _NSUB - 1)
        def _():
            pltpu.sync_copy(
                acc.at[pl.ds(15 * _WPT, _WLAST)],
                out.at[cid, pl.ds(15 * _WPT, _WLAST)],
            )

    return edge


_HIST = _hist_kernel()
_EDGE = [_edge_kernel(l) for l in range(_L)]


def kernel(x, edge_index, edge_attr, W_in, b_in, Ws_msg, bs_msg, W_out, b_out):
    f32 = jnp.float32
    pad = _EP - _E
    src = edge_index[0].astype(jnp.int32)
    dst = edge_index[1].astype(jnp.int32)
    srcp = jnp.pad(src, (0, pad)).reshape(_NROWS, _CH)
    dstp = jnp.pad(dst, (0, pad)).reshape(_NROWS, _CH)
    dsth = jnp.pad(dst, (0, pad), constant_values=_N).reshape(_NROWS, _CH)
    Z = jnp.zeros((_N2, _HALF), f32)
    Z16 = jnp.zeros((_N2, 16), f32)
    ones = jnp.ones((_CH, 16), f32)

    xp = jnp.pad(x.astype(f32), ((0, 0), (0, 8 - x.shape[1])))
    W_in_p = jnp.pad(W_in, ((0, 8 - W_in.shape[0]), (0, 0)))
    Wcr = Ws_msg[:, 2 * _HID:, :].reshape(_L, 4, 2, _HALF)
    W16 = jnp.einsum("qp,lkcf->lcqkpf", jnp.eye(16, dtype=f32), Wcr).reshape(_L, 2, 64, 512)
    b16 = jnp.tile(bs_msg.reshape(_L, 2, 1, _HALF), (1, 1, 1, 16))
    ea16 = jnp.concatenate([edge_attr[q::16] for q in range(16)], axis=1)
    Wa = Ws_msg[:, :_HID, :].reshape(_L, _HID, 2, _HALF).transpose(0, 2, 1, 3)
    Wb = Ws_msg[:, _HID:2 * _HID, :].reshape(_L, _HID, 2, _HALF).transpose(0, 2, 1, 3)

    h = _h_in(xp, W_in_p, b_in.reshape(1, _HID))
    C = _c_all(ea16, W16, b16).reshape(_L, 2, _EP, _HALF)
    cnt = _HIST(dsth, Z16, ones)
    A, Bm = _ab_first(h, Wa[0], Wb[0])
    for l in range(_L):
        agg = _EDGE[l](srcp, dstp, A, Bm, C, Z)
        if l + 1 < _L:
            h, A, Bm = _ab_next(h, agg, cnt, Wa[l + 1], Wb[l + 1])
        else:
            return _out_proj(h, agg, cnt, W_out, b_out.reshape(1, _OUT))


# speedup vs baseline: 1.1173x; 1.1173x over previous
import functools

import jax
import jax.numpy as jnp
from jax import lax
from jax.experimental import pallas as pl
from jax.experimental.pallas import tpu as pltpu
from jax.experimental.pallas import tpu_sc as plsc

_N = 50000
_E = 800000
_HID = 64
_HALF = 32
_OUT = 128
_L = 4
_CH = 128
_SUP = 2
_NSUB = 16
_NCHUNK = 392
_NSUP = _NCHUNK // _SUP
_EPT = _NCHUNK * _CH
_EP = _NSUB * _EPT
_NROWS = _EP // _CH
_N2 = 50048
_RPT = _N2 // _NSUB
_WPT = 3128
_WLAST = _N - 15 * _WPT
_BN = 2000
_BE = 2048

_MESH = plsc.VectorSubcoreMesh(core_axis_name="c", subcore_axis_name="s")
_SC_PARAMS = pltpu.CompilerParams(use_tc_tiling_on_sc=False)


def _in_body(xb, wb, bb, ob):
    ob[...] = jnp.dot(xb[...], wb[...], preferred_element_type=jnp.float32) + bb[...]


def _h_in(xp, W, b):
    return pl.pallas_call(
        _in_body,
        grid=(_N // _BN,),
        in_specs=[
            pl.BlockSpec((_BN, 8), lambda i: (i, 0)),
            pl.BlockSpec((8, _HID), lambda i: (0, 0)),
            pl.BlockSpec((1, _HID), lambda i: (0, 0)),
        ],
        out_specs=pl.BlockSpec((_BN, _HID), lambda i: (i, 0)),
        out_shape=jax.ShapeDtypeStruct((_N, _HID), jnp.float32),
    )(xp, W, b)


_E4 = _E // 4
_EP4 = _EP // 4
_BE4 = 2048


def _c_body(eb, wb, bb, ob):
    i = pl.program_id(0)
    rows = i * _BE4 + lax.broadcasted_iota(jnp.int32, (_BE4, 1), 0)
    m = jnp.dot(eb[...], wb[0], preferred_element_type=jnp.float32) + bb[0]
    ob[...] = jnp.where(rows < _E4, m, -1e30).reshape(_BE4 * 128)


def _c_layer(ea4, W4l, b4l):
    nblk = _EP4 // _BE4
    return pl.pallas_call(
        _c_body,
        grid=(nblk, 2),
        in_specs=[
            pl.BlockSpec((_BE4, 16), lambda e, c: (jnp.minimum(e, _E4 // _BE4), 0)),
            pl.BlockSpec((1, 16, 128), lambda e, c: (c, 0, 0)),
            pl.BlockSpec((1, 1, 128), lambda e, c: (c, 0, 0)),
        ],
        out_specs=pl.BlockSpec((_BE4 * 128,), lambda e, c: (c * nblk + e,)),
        out_shape=jax.ShapeDtypeStruct((2 * _EP * _HALF,), jnp.float32),
    )(ea4, W4l, b4l)


def _ab_first_body(hb, wab, wbb, ao, bo):
    h = hb[...]
    ao[...] = jnp.dot(h, wab[0], preferred_element_type=jnp.float32)
    bo[...] = jnp.dot(h, wbb[0], preferred_element_type=jnp.float32)


def _ab_first(h, Wa, Wb):
    nb = _N // _BN
    return pl.pallas_call(
        _ab_first_body,
        grid=(2, nb),
        in_specs=[
            pl.BlockSpec((_BN, _HID), lambda c, i: (i, 0)),
            pl.BlockSpec((1, _HID, _HALF), lambda c, i: (c, 0, 0)),
            pl.BlockSpec((1, _HID, _HALF), lambda c, i: (c, 0, 0)),
        ],
        out_specs=[
            pl.BlockSpec((_BN, _HALF), lambda c, i: (c * nb + i, 0)),
            pl.BlockSpec((_BN, _HALF), lambda c, i: (c * nb + i, 0)),
        ],
        out_shape=[
            jax.ShapeDtypeStruct((2 * _N, _HALF), jnp.float32),
            jax.ShapeDtypeStruct((2 * _N, _HALF), jnp.float32),
        ],
    )(h, Wa, Wb)


def _ab_body(hb, gb, cb, wab, wbb, ho, ao, bo):
    inv = 1.0 / jnp.maximum(cb[...][:, :1], 1.0)
    hn = hb[...] + jnp.concatenate([gb[0], gb[1]], axis=1) * inv
    ho[...] = hn
    ao[...] = jnp.dot(hn, wab[0], preferred_element_type=jnp.float32)
    bo[...] = jnp.dot(hn, wbb[0], preferred_element_type=jnp.float32)


def _ab_next(h, agg, cnt, Wa, Wb):
    nb = _N // _BN
    return pl.pallas_call(
        _ab_body,
        grid=(2, nb),
        in_specs=[
            pl.BlockSpec((_BN, _HID), lambda c, i: (i, 0)),
            pl.BlockSpec((2, _BN, _HALF), lambda c, i: (0, i, 0)),
            pl.BlockSpec((_BN, 16), lambda c, i: (i, 0)),
            pl.BlockSpec((1, _HID, _HALF), lambda c, i: (c, 0, 0)),
            pl.BlockSpec((1, _HID, _HALF), lambda c, i: (c, 0, 0)),
        ],
        out_specs=[
            pl.BlockSpec((_BN, _HID), lambda c, i: (i, 0)),
            pl.BlockSpec((_BN, _HALF), lambda c, i: (c * nb + i, 0)),
            pl.BlockSpec((_BN, _HALF), lambda c, i: (c * nb + i, 0)),
        ],
        out_shape=[
            jax.ShapeDtypeStruct((_N, _HID), jnp.float32),
            jax.ShapeDtypeStruct((2 * _N, _HALF), jnp.float32),
            jax.ShapeDtypeStruct((2 * _N, _HALF), jnp.float32),
        ],
    )(h, agg, cnt, Wa, Wb)


def _out_body(hb, gb, cb, wb, bb, ob):
    inv = 1.0 / jnp.maximum(cb[...][:, :1], 1.0)
    hn = hb[...] + jnp.concatenate([gb[0], gb[1]], axis=1) * inv
    ob[...] = jnp.dot(hn, wb[...], preferred_element_type=jnp.float32) + bb[...]


def _out_proj(h, agg, cnt, W, b):
    return pl.pallas_call(
        _out_body,
        grid=(_N // _BN,),
        in_specs=[
            pl.BlockSpec((_BN, _HID), lambda i: (i, 0)),
            pl.BlockSpec((2, _BN, _HALF), lambda i: (0, i, 0)),
            pl.BlockSpec((_BN, 16), lambda i: (i, 0)),
            pl.BlockSpec((_HID, _OUT), lambda i: (0, 0)),
            pl.BlockSpec((1, _OUT), lambda i: (0, 0)),
        ],
        out_specs=pl.BlockSpec((_BN, _OUT), lambda i: (i, 0)),
        out_shape=jax.ShapeDtypeStruct((_N, _OUT), jnp.float32),
    )(h, agg, cnt, W, b)


def _hist_kernel():
    @functools.partial(
        pl.kernel,
        out_type=jax.ShapeDtypeStruct((_N, 16), jnp.float32),
        mesh=_MESH,
        compiler_params=_SC_PARAMS,
        scratch_types=[
            pltpu.VMEM((_SUP, _CH), jnp.int32),
            pltpu.VMEM((_CH, 16), jnp.float32),
            pltpu.VMEM_SHARED((_N2, 16), jnp.float32),
            pltpu.SemaphoreType.DMA,
        ],
    )
    def hist(dsth, Z16, ones, out, di, ones_v, accc, ss):
        cid = lax.axis_index("c")
        sid = lax.axis_index("s")

        @pl.when(cid == 0)
        def _():
            pltpu.sync_copy(
                Z16.at[pl.ds(sid * _RPT, _RPT)],
                accc.at[pl.ds(sid * _RPT, _RPT)],
            )
            pltpu.sync_copy(ones, ones_v)
            plsc.subcore_barrier()

            def step(u, carry):
                r0 = sid * _NCHUNK + u * _SUP
                pltpu.sync_copy(dsth.at[pl.ds(r0, _SUP)], di)
                cps = [
                    pltpu.async_copy(ones_v, accc.at[di.at[j]], ss, add=True)
                    for j in range(_SUP)
                ]
                for g in cps:
                    g.wait()
                return carry

            lax.fori_loop(0, _NSUP, step, 0)
            plsc.subcore_barrier()

            @pl.when(sid < _NSUB - 1)
            def _():
                pltpu.sync_copy(
                    accc.at[pl.ds(sid * _WPT, _WPT)],
                    out.at[pl.ds(sid * _WPT, _WPT)],
                )

            @pl.when(sid == _NSUB - 1)
            def _():
                pltpu.sync_copy(
                    accc.at[pl.ds(15 * _WPT, _WLAST)],
                    out.at[pl.ds(15 * _WPT, _WLAST)],
                )

    return hist


def _edge_kernel():
    @functools.partial(
        pl.kernel,
        out_type=jax.ShapeDtypeStruct((2, _N, _HALF), jnp.float32),
        mesh=_MESH,
        compiler_params=_SC_PARAMS,
        scratch_types=[
            pltpu.VMEM((_SUP, _CH), jnp.int32),
            pltpu.VMEM((_SUP, _CH), jnp.int32),
            pltpu.VMEM((_SUP, _CH), jnp.int32),
            pltpu.VMEM((_SUP, _CH), jnp.int32),
            pltpu.VMEM((_SUP, _CH), jnp.int32),
            pltpu.VMEM((_SUP, _CH), jnp.int32),
            pltpu.VMEM((_SUP * _CH, _HALF), jnp.float32),
            pltpu.VMEM((_SUP * _CH, _HALF), jnp.float32),
            pltpu.VMEM_SHARED((_N2, _HALF), jnp.float32),
            pltpu.SemaphoreType.DMA,
            pltpu.SemaphoreType.DMA,
            pltpu.SemaphoreType.DMA,
            pltpu.SemaphoreType.DMA,
            pltpu.SemaphoreType.DMA,
            pltpu.SemaphoreType.DMA,
            pltpu.SemaphoreType.DMA,
            pltpu.SemaphoreType.DMA,
        ],
    )
    def edge(srcp, dstp, A, B, C, Z, out,
             si0, si1, di0, di1, bi0, bi1, c0, c1, acc,
             mi0, mi1, mc0, mc1, mg0, mg1, ms0, ms1):
        cid = lax.axis_index("c")
        sid = lax.axis_index("s")
        SI, DI, BI = (si0, si1), (di0, di1), (bi0, bi1)
        CS = (c0, c1)
        MI, MC, MG, MS = (mi0, mi1), (mc0, mc1), (mg0, mg1), (ms0, ms1)
        off = jnp.full((16,), cid * _N, jnp.int32)

        pltpu.sync_copy(Z.at[pl.ds(sid * _RPT, _RPT)], acc.at[pl.ds(sid * _RPT, _RPT)])
        plsc.subcore_barrier()

        def issue_loads(i, b):
            r0 = sid * _NCHUNK + i * _SUP
            pltpu.async_copy(srcp.at[pl.ds(r0, _SUP)], SI[b], MI[b])
            pltpu.async_copy(dstp.at[pl.ds(r0, _SUP)], DI[b], MI[b])
            pltpu.async_copy(C.at[cid, pl.ds(r0 * _CH, _SUP * _CH)], CS[b], MC[b])

        def wait_loads(i, b):
            r0 = sid * _NCHUNK + i * _SUP
            pltpu.make_async_copy(srcp.at[pl.ds(r0, _SUP)], SI[b], MI[b]).wait()
            pltpu.make_async_copy(dstp.at[pl.ds(r0, _SUP)], DI[b], MI[b]).wait()
            pltpu.make_async_copy(C.at[cid, pl.ds(r0 * _CH, _SUP * _CH)], CS[b], MC[b]).wait()

        def offs(b):
            for k in range(_SUP):
                for g in range(_CH // 16):
                    s = pl.ds(g * 16, 16)
                    BI[b][k, s] = DI[b][k, s] + off
                    SI[b][k, s] = SI[b][k, s] + off

        def issue_gathers(b):
            for j in range(_SUP):
                pltpu.async_copy(A.at[SI[b].at[j]], CS[b].at[pl.ds(j * _CH, _CH)], MG[b], add=True)
                pltpu.async_copy(B.at[BI[b].at[j]], CS[b].at[pl.ds(j * _CH, _CH)], MG[b], add=True)

        def wait_gathers(b):
            for j in range(_SUP):
                pltpu.make_async_copy(A.at[SI[b].at[j]], CS[b].at[pl.ds(j * _CH, _CH)], MG[b]).wait()
                pltpu.make_async_copy(B.at[BI[b].at[j]], CS[b].at[pl.ds(j * _CH, _CH)], MG[b]).wait()

        def relu(b):
            def rows(rb, c2):
                for t in range(4):
                    r = rb * 4 + t
                    for f in (0, 16):
                        s = pl.ds(f, 16)
                        CS[b][r, s] = jnp.maximum(CS[b][r, s], 0.0)
                return c2

            lax.fori_loop(0, (_SUP * _CH) // 4, rows, 0)

        def issue_scatter(b):
            for j in range(_SUP):
                pltpu.async_copy(CS[b].at[pl.ds(j * _CH, _CH)], acc.at[DI[b].at[j]], MS[b], add=True)

        def wait_scatter(b):
            for j in range(_SUP):
                pltpu.make_async_copy(CS[b].at[pl.ds(j * _CH, _CH)], acc.at[DI[b].at[j]], MS[b]).wait()

        issue_loads(0, 0)
        issue_loads(1, 1)
        wait_loads(0, 0)
        offs(0)
        issue_gathers(0)

        def body(i2, carry):
            i = i2 * 2
            wait_loads(i + 1, 1)
            offs(1)
            issue_gathers(1)

            wait_gathers(0)
            relu(0)
            issue_scatter(0)
            wait_scatter(0)

            @pl.when(i + 2 < _NSUP)
            def _():
                issue_loads(i + 2, 0)

            wait_gathers(1)
            relu(1)
            issue_scatter(1)
            wait_scatter(1)

            @pl.when(i + 3 < _NSUP)
            def _():
                issue_loads(i + 3, 1)

            @pl.when(i + 2 < _NSUP)
            def _():
                wait_loads(i + 2, 0)
                offs(0)
                issue_gathers(0)

            return carry

        lax.fori_loop(0, _NSUP // 2, body, 0)
        plsc.subcore_barrier()

        @pl.when(sid < _NSUB - 1)
        def _():
            pltpu.sync_copy(
                acc.at[pl.ds(sid * _WPT, _WPT)],
                out.at[cid, pl.ds(sid * _WPT, _WPT)],
            )

        @pl.when(sid == _NSUB - 1)
        def _():
            pltpu.sync_copy(
                acc.at[pl.ds(15 * _WPT, _WLAST)],
                out.at[cid, pl.ds(15 * _WPT, _WLAST)],
            )

    return edge


_HIST = _hist_kernel()
_EDGE = _edge_kernel()


def kernel(x, edge_index, edge_attr, W_in, b_in, Ws_msg, bs_msg, W_out, b_out):
    f32 = jnp.float32
    pad = _EP - _E
    src = edge_index[0].astype(jnp.int32)
    dst = edge_index[1].astype(jnp.int32)
    srcp = jnp.pad(src, (0, pad)).reshape(_NROWS, _CH)
    dstp = jnp.pad(dst, (0, pad)).reshape(_NROWS, _CH)
    dsth = jnp.pad(dst, (0, pad), constant_values=_N).reshape(_NROWS, _CH)
    Z = jnp.zeros((_N2, _HALF), f32)
    Z16 = jnp.zeros((_N2, 16), f32)
    ones = jnp.ones((_CH, 16), f32)

    xp = jnp.pad(x.astype(f32), ((0, 0), (0, 8 - x.shape[1])))
    W_in_p = jnp.pad(W_in, ((0, 8 - W_in.shape[0]), (0, 0)))
    Wcr = Ws_msg[:, 2 * _HID:, :].reshape(_L, 4, 2, _HALF)
    W4 = jnp.einsum("qp,lkcf->lcqkpf", jnp.eye(4, dtype=f32), Wcr).reshape(_L, 2, 16, 128)
    b4 = jnp.tile(bs_msg.reshape(_L, 2, 1, _HALF), (1, 1, 1, 4))
    ea4 = jnp.concatenate([edge_attr[q::4] for q in range(4)], axis=1)
    Wa = Ws_msg[:, :_HID, :].reshape(_L, _HID, 2, _HALF).transpose(0, 2, 1, 3)
    Wb = Ws_msg[:, _HID:2 * _HID, :].reshape(_L, _HID, 2, _HALF).transpose(0, 2, 1, 3)

    h = _h_in(xp, W_in_p, b_in.reshape(1, _HID))
    Cs = [_c_layer(ea4, W4[l], b4[l]).reshape(2, _EP, _HALF) for l in range(_L)]
    cnt = _HIST(dsth, Z16, ones)
    A, Bm = _ab_first(h, Wa[0], Wb[0])
    for l in range(_L):
        agg = _EDGE(srcp, dstp, A, Bm, Cs[l], Z)
        if l + 1 < _L:
            h, A, Bm = _ab_next(h, agg, cnt, Wa[l + 1], Wb[l + 1])
        else:
            return _out_proj(h, agg, cnt, W_out, b_out.reshape(1, _OUT))


# speedup vs baseline: 1.1686x; 1.0460x over previous
import functools

import jax
import jax.numpy as jnp
from jax import lax
from jax.experimental import pallas as pl
from jax.experimental.pallas import tpu as pltpu
from jax.experimental.pallas import tpu_sc as plsc

_N = 50000
_E = 800000
_HID = 64
_HALF = 32
_OUT = 128
_L = 4
_CH = 128
_SUP = 2
_NSUB = 16
_NCHUNK = 392
_NSUP = _NCHUNK // _SUP
_EPT = _NCHUNK * _CH
_EP = _NSUB * _EPT
_NROWS = _EP // _CH
_N2 = 50048
_RPT = _N2 // _NSUB
_WPT = 3128
_WLAST = _N - 15 * _WPT
_BN = 2000
_BE = 2048

_MESH = plsc.VectorSubcoreMesh(core_axis_name="c", subcore_axis_name="s")
_SC_PARAMS = pltpu.CompilerParams(use_tc_tiling_on_sc=False)


def _in_body(xb, wb, bb, ob):
    ob[...] = jnp.dot(xb[...], wb[...], preferred_element_type=jnp.float32) + bb[...]


def _h_in(xp, W, b):
    return pl.pallas_call(
        _in_body,
        grid=(_N // _BN,),
        in_specs=[
            pl.BlockSpec((_BN, 8), lambda i: (i, 0)),
            pl.BlockSpec((8, _HID), lambda i: (0, 0)),
            pl.BlockSpec((1, _HID), lambda i: (0, 0)),
        ],
        out_specs=pl.BlockSpec((_BN, _HID), lambda i: (i, 0)),
        out_shape=jax.ShapeDtypeStruct((_N, _HID), jnp.float32),
    )(xp, W, b)


_E4 = _E // 4
_EP4 = _EP // 4
_BE4 = 2048


def _c_body(eb, wb, bb, ob):
    i = pl.program_id(0)
    rows = i * _BE4 + lax.broadcasted_iota(jnp.int32, (_BE4, 1), 0)
    m = jnp.dot(eb[...], wb[0], preferred_element_type=jnp.float32) + bb[0]
    ob[...] = jnp.where(rows < _E4, m, -1e30).reshape(_BE4 * 128)


def _c_layer(ea4, W4l, b4l):
    nblk = _EP4 // _BE4
    return pl.pallas_call(
        _c_body,
        grid=(nblk, 2),
        in_specs=[
            pl.BlockSpec((_BE4, 16), lambda e, c: (jnp.minimum(e, _E4 // _BE4), 0)),
            pl.BlockSpec((1, 16, 128), lambda e, c: (c, 0, 0)),
            pl.BlockSpec((1, 1, 128), lambda e, c: (c, 0, 0)),
        ],
        out_specs=pl.BlockSpec((_BE4 * 128,), lambda e, c: (c * nblk + e,)),
        out_shape=jax.ShapeDtypeStruct((2 * _EP * _HALF,), jnp.float32),
    )(ea4, W4l, b4l)


def _ab_first_body(hb, wab, wbb, ao, bo):
    h = hb[...]
    ao[...] = jnp.dot(h, wab[0], preferred_element_type=jnp.float32)
    bo[...] = jnp.dot(h, wbb[0], preferred_element_type=jnp.float32)


def _ab_first(h, Wa, Wb):
    nb = _N // _BN
    return pl.pallas_call(
        _ab_first_body,
        grid=(2, nb),
        in_specs=[
            pl.BlockSpec((_BN, _HID), lambda c, i: (i, 0)),
            pl.BlockSpec((1, _HID, _HALF), lambda c, i: (c, 0, 0)),
            pl.BlockSpec((1, _HID, _HALF), lambda c, i: (c, 0, 0)),
        ],
        out_specs=[
            pl.BlockSpec((_BN, _HALF), lambda c, i: (c * nb + i, 0)),
            pl.BlockSpec((_BN, _HALF), lambda c, i: (c * nb + i, 0)),
        ],
        out_shape=[
            jax.ShapeDtypeStruct((2 * _N, _HALF), jnp.float32),
            jax.ShapeDtypeStruct((2 * _N, _HALF), jnp.float32),
        ],
    )(h, Wa, Wb)


def _ab_body(hb, gb, cb, wab, wbb, ho, ao, bo):
    inv = 1.0 / jnp.maximum(cb[...][:, :1], 1.0)
    hn = hb[...] + jnp.concatenate([gb[0], gb[1]], axis=1) * inv
    ho[...] = hn
    ao[...] = jnp.dot(hn, wab[0], preferred_element_type=jnp.float32)
    bo[...] = jnp.dot(hn, wbb[0], preferred_element_type=jnp.float32)


def _ab_next(h, agg, cnt, Wa, Wb):
    nb = _N // _BN
    return pl.pallas_call(
        _ab_body,
        grid=(2, nb),
        in_specs=[
            pl.BlockSpec((_BN, _HID), lambda c, i: (i, 0)),
            pl.BlockSpec((2, _BN, _HALF), lambda c, i: (0, i, 0)),
            pl.BlockSpec((_BN, 16), lambda c, i: (i, 0)),
            pl.BlockSpec((1, _HID, _HALF), lambda c, i: (c, 0, 0)),
            pl.BlockSpec((1, _HID, _HALF), lambda c, i: (c, 0, 0)),
        ],
        out_specs=[
            pl.BlockSpec((_BN, _HID), lambda c, i: (i, 0)),
            pl.BlockSpec((_BN, _HALF), lambda c, i: (c * nb + i, 0)),
            pl.BlockSpec((_BN, _HALF), lambda c, i: (c * nb + i, 0)),
        ],
        out_shape=[
            jax.ShapeDtypeStruct((_N, _HID), jnp.float32),
            jax.ShapeDtypeStruct((2 * _N, _HALF), jnp.float32),
            jax.ShapeDtypeStruct((2 * _N, _HALF), jnp.float32),
        ],
    )(h, agg, cnt, Wa, Wb)


def _out_body(hb, gb, cb, wb, bb, ob):
    inv = 1.0 / jnp.maximum(cb[...][:, :1], 1.0)
    hn = hb[...] + jnp.concatenate([gb[0], gb[1]], axis=1) * inv
    ob[...] = jnp.dot(hn, wb[...], preferred_element_type=jnp.float32) + bb[...]


def _out_proj(h, agg, cnt, W, b):
    return pl.pallas_call(
        _out_body,
        grid=(_N // _BN,),
        in_specs=[
            pl.BlockSpec((_BN, _HID), lambda i: (i, 0)),
            pl.BlockSpec((2, _BN, _HALF), lambda i: (0, i, 0)),
            pl.BlockSpec((_BN, 16), lambda i: (i, 0)),
            pl.BlockSpec((_HID, _OUT), lambda i: (0, 0)),
            pl.BlockSpec((1, _OUT), lambda i: (0, 0)),
        ],
        out_specs=pl.BlockSpec((_BN, _OUT), lambda i: (i, 0)),
        out_shape=jax.ShapeDtypeStruct((_N, _OUT), jnp.float32),
    )(h, agg, cnt, W, b)


def _hist_kernel():
    @functools.partial(
        pl.kernel,
        out_type=jax.ShapeDtypeStruct((_N, 16), jnp.float32),
        mesh=_MESH,
        compiler_params=_SC_PARAMS,
        scratch_types=[
            pltpu.VMEM((_SUP, _CH), jnp.int32),
            pltpu.VMEM((_CH, 16), jnp.float32),
            pltpu.VMEM_SHARED((_N2, 16), jnp.float32),
            pltpu.SemaphoreType.DMA,
        ],
    )
    def hist(dsth, Z16, ones, out, di, ones_v, accc, ss):
        cid = lax.axis_index("c")
        sid = lax.axis_index("s")

        @pl.when(cid == 0)
        def _():
            pltpu.sync_copy(
                Z16.at[pl.ds(sid * _RPT, _RPT)],
                accc.at[pl.ds(sid * _RPT, _RPT)],
            )
            pltpu.sync_copy(ones, ones_v)
            plsc.subcore_barrier()

            def step(u, carry):
                r0 = sid * _NCHUNK + u * _SUP
                pltpu.sync_copy(dsth.at[pl.ds(r0, _SUP)], di)
                cps = [
                    pltpu.async_copy(ones_v, accc.at[di.at[j]], ss, add=True)
                    for j in range(_SUP)
                ]
                for g in cps:
                    g.wait()
                return carry

            lax.fori_loop(0, _NSUP, step, 0)
            plsc.subcore_barrier()

            @pl.when(sid < _NSUB - 1)
            def _():
                pltpu.sync_copy(
                    accc.at[pl.ds(sid * _WPT, _WPT)],
                    out.at[pl.ds(sid * _WPT, _WPT)],
                )

            @pl.when(sid == _NSUB - 1)
            def _():
                pltpu.sync_copy(
                    accc.at[pl.ds(15 * _WPT, _WLAST)],
                    out.at[pl.ds(15 * _WPT, _WLAST)],
                )

    return hist


def _edge_kernel():
    @functools.partial(
        pl.kernel,
        out_type=jax.ShapeDtypeStruct((2, _N, _HALF), jnp.float32),
        mesh=_MESH,
        compiler_params=_SC_PARAMS,
        scratch_types=[
            pltpu.VMEM((_SUP, _CH), jnp.int32),
            pltpu.VMEM((_SUP, _CH), jnp.int32),
            pltpu.VMEM((_SUP, _CH), jnp.int32),
            pltpu.VMEM((_SUP, _CH), jnp.int32),
            pltpu.VMEM((_SUP, _CH), jnp.int32),
            pltpu.VMEM((_SUP, _CH), jnp.int32),
            pltpu.VMEM((_SUP * _CH, _HALF), jnp.float32),
            pltpu.VMEM((_SUP * _CH, _HALF), jnp.float32),
            pltpu.VMEM_SHARED((_N2, _HALF), jnp.float32),
            pltpu.SemaphoreType.DMA,
            pltpu.SemaphoreType.DMA,
            pltpu.SemaphoreType.DMA,
            pltpu.SemaphoreType.DMA,
            pltpu.SemaphoreType.DMA,
            pltpu.SemaphoreType.DMA,
            pltpu.SemaphoreType.DMA,
            pltpu.SemaphoreType.DMA,
        ],
    )
    def edge(srcp, dstp, A, B, C, Z, out,
             si0, si1, di0, di1, bi0, bi1, c0, c1, acc,
             mi0, mi1, mc0, mc1, mg0, mg1, ms0, ms1):
        cid = lax.axis_index("c")
        sid = lax.axis_index("s")
        SI, DI, BI = (si0, si1), (di0, di1), (bi0, bi1)
        CS = (c0, c1)
        MI, MC, MG, MS = (mi0, mi1), (mc0, mc1), (mg0, mg1), (ms0, ms1)
        off = jnp.full((16,), cid * _N, jnp.int32)

        pltpu.sync_copy(Z.at[pl.ds(sid * _RPT, _RPT)], acc.at[pl.ds(sid * _RPT, _RPT)])
        plsc.subcore_barrier()

        def issue_loads(i, b):
            r0 = sid * _NCHUNK + i * _SUP
            pltpu.async_copy(srcp.at[pl.ds(r0, _SUP)], SI[b], MI[b])
            pltpu.async_copy(dstp.at[pl.ds(r0, _SUP)], DI[b], MI[b])
            pltpu.async_copy(C.at[cid, pl.ds(r0 * _CH, _SUP * _CH)], CS[b], MC[b])

        def wait_loads(i, b):
            r0 = sid * _NCHUNK + i * _SUP
            pltpu.make_async_copy(srcp.at[pl.ds(r0, _SUP)], SI[b], MI[b]).wait()
            pltpu.make_async_copy(dstp.at[pl.ds(r0, _SUP)], DI[b], MI[b]).wait()
            pltpu.make_async_copy(C.at[cid, pl.ds(r0 * _CH, _SUP * _CH)], CS[b], MC[b]).wait()

        def offs(b):
            for k in range(_SUP):
                for g in range(_CH // 16):
                    s = pl.ds(g * 16, 16)
                    BI[b][k, s] = DI[b][k, s] + off
                    SI[b][k, s] = SI[b][k, s] + off

        def issue_gathers(b):
            for j in range(_SUP):
                pltpu.async_copy(A.at[SI[b].at[j]], CS[b].at[pl.ds(j * _CH, _CH)], MG[b], add=True)
                pltpu.async_copy(B.at[BI[b].at[j]], CS[b].at[pl.ds(j * _CH, _CH)], MG[b], add=True)

        def wait_gathers(b):
            for j in range(_SUP):
                pltpu.make_async_copy(A.at[SI[b].at[j]], CS[b].at[pl.ds(j * _CH, _CH)], MG[b]).wait()
                pltpu.make_async_copy(B.at[BI[b].at[j]], CS[b].at[pl.ds(j * _CH, _CH)], MG[b]).wait()

        def relu(b):
            def rows(rb, c2):
                for t in range(4):
                    r = rb * 4 + t
                    for f in (0, 16):
                        s = pl.ds(f, 16)
                        CS[b][r, s] = jnp.maximum(CS[b][r, s], 0.0)
                return c2

            lax.fori_loop(0, (_SUP * _CH) // 4, rows, 0)

        def issue_scatter(b):
            for j in range(_SUP):
                pltpu.async_copy(CS[b].at[pl.ds(j * _CH, _CH)], acc.at[DI[b].at[j]], MS[b], add=True)

        def wait_scatter(b):
            for j in range(_SUP):
                pltpu.make_async_copy(CS[b].at[pl.ds(j * _CH, _CH)], acc.at[DI[b].at[j]], MS[b]).wait()

        issue_loads(0, 0)
        issue_loads(1, 1)
        wait_loads(0, 0)
        offs(0)
        issue_gathers(0)

        def body(i2, carry):
            i = i2 * 2
            wait_loads(i + 1, 1)
            offs(1)
            issue_gathers(1)

            wait_gathers(0)
            relu(0)
            issue_scatter(0)
            wait_scatter(0)

            @pl.when(i + 2 < _NSUP)
            def _():
                issue_loads(i + 2, 0)

            wait_gathers(1)
            relu(1)
            issue_scatter(1)
            wait_scatter(1)

            @pl.when(i + 3 < _NSUP)
            def _():
                issue_loads(i + 3, 1)

            @pl.when(i + 2 < _NSUP)
            def _():
                wait_loads(i + 2, 0)
                offs(0)
                issue_gathers(0)

            return carry

        lax.fori_loop(0, _NSUP // 2, body, 0)
        plsc.subcore_barrier()

        @pl.when(sid < _NSUB - 1)
        def _():
            pltpu.sync_copy(
                acc.at[pl.ds(sid * _WPT, _WPT)],
                out.at[cid, pl.ds(sid * _WPT, _WPT)],
            )

        @pl.when(sid == _NSUB - 1)
        def _():
            pltpu.sync_copy(
                acc.at[pl.ds(15 * _WPT, _WLAST)],
                out.at[cid, pl.ds(15 * _WPT, _WLAST)],
            )

    return edge


_HIST = _hist_kernel()
_EDGE = _edge_kernel()


def kernel(x, edge_index, edge_attr, W_in, b_in, Ws_msg, bs_msg, W_out, b_out):
    f32 = jnp.float32
    pad = _EP - _E
    src = edge_index[0].astype(jnp.int32)
    dst = edge_index[1].astype(jnp.int32)
    srcp = jnp.pad(src, (0, pad)).reshape(_NROWS, _CH)
    dstp = jnp.pad(dst, (0, pad)).reshape(_NROWS, _CH)
    dsth = jnp.pad(dst, (0, pad), constant_values=_N).reshape(_NROWS, _CH)
    Z = jnp.zeros((_N2, _HALF), f32)
    Z16 = jnp.zeros((_N2, 16), f32)
    ones = jnp.ones((_CH, 16), f32)

    xp = jnp.pad(x.astype(f32), ((0, 0), (0, 8 - x.shape[1])))
    W_in_p = jnp.pad(W_in, ((0, 8 - W_in.shape[0]), (0, 0)))
    Wcr = Ws_msg[:, 2 * _HID:, :].reshape(_L, 4, 2, _HALF)
    W4 = jnp.einsum("qp,lkcf->lcqkpf", jnp.eye(4, dtype=f32), Wcr).reshape(_L, 2, 16, 128)
    b4 = jnp.tile(bs_msg.reshape(_L, 2, 1, _HALF), (1, 1, 1, 4))
    ea4 = edge_attr.T.reshape(4, _E4, 4).transpose(1, 2, 0).reshape(_E4, 16)
    Wa = Ws_msg[:, :_HID, :].reshape(_L, _HID, 2, _HALF).transpose(0, 2, 1, 3)
    Wb = Ws_msg[:, _HID:2 * _HID, :].reshape(_L, _HID, 2, _HALF).transpose(0, 2, 1, 3)

    h = _h_in(xp, W_in_p, b_in.reshape(1, _HID))
    Cs = [_c_layer(ea4, W4[l], b4[l]).reshape(2, _EP, _HALF) for l in range(_L)]
    cnt = _HIST(dsth, Z16, ones)
    A, Bm = _ab_first(h, Wa[0], Wb[0])
    for l in range(_L):
        agg = _EDGE(srcp, dstp, A, Bm, Cs[l], Z)
        if l + 1 < _L:
            h, A, Bm = _ab_next(h, agg, cnt, Wa[l + 1], Wb[l + 1])
        else:
            return _out_proj(h, agg, cnt, W_out, b_out.reshape(1, _OUT))
